# baseline (speedup 1.0000x reference)
import jax
import jax.numpy as jnp
from jax.experimental import pallas as pl
from jax.experimental.pallas import tpu as pltpu

_BLOCK = 5000


def _fused_head_kernel(feat_ref, w1_ref, b1_ref, gain_ref, beta_ref,
                       w2_ref, b2_ref,
                       lw1_ref, lb1_ref, lw2_ref, lb2_ref,
                       sw1_ref, sb1_ref, sw2_ref, sb2_ref,
                       cw1_ref, cb1_ref, cw2_ref, cb2_ref,
                       fw1_ref, fb1_ref, fw2_ref, fb2_ref,
                       boxes_ref, scales_ref, ctx_ref, conf_ref):
    min_size, max_size = 0.02, 0.1

    def dot(a, b):
        return jnp.dot(a, b, preferred_element_type=jnp.float32)

    x = dot(feat_ref[...], w1_ref[...]) + b1_ref[...]
    mu = jnp.mean(x, axis=-1, keepdims=True)
    var = jnp.mean(jnp.square(x), axis=-1, keepdims=True) - jnp.square(mu)
    x = (x - mu) * jax.lax.rsqrt(var + 1e-5)
    x = jnp.maximum(x * gain_ref[...] + beta_ref[...], 0.0)

    x = dot(x, w2_ref[...]) + b2_ref[...]

    boxes = jax.nn.sigmoid(
        dot(jnp.maximum(dot(x, lw1_ref[...]) + lb1_ref[...], 0.0),
            lw2_ref[...]) + lb2_ref[...])
    scales = jax.nn.sigmoid(
        dot(jnp.maximum(dot(x, sw1_ref[...]) + sb1_ref[...], 0.0),
            sw2_ref[...]) + sb2_ref[...]) * (max_size - min_size) + min_size
    ctx = jax.nn.sigmoid(
        dot(jnp.maximum(dot(x, cw1_ref[...]) + cb1_ref[...], 0.0),
            cw2_ref[...]) + cb2_ref[...])

    combined = jnp.concatenate([boxes, scales, ctx], axis=-1)
    hf = jnp.maximum(
        dot(x, fw1_ref[0:256, :]) + dot(combined, fw1_ref[256:263, :])
        + fb1_ref[...], 0.0)
    conf = jax.nn.sigmoid(dot(hf, fw2_ref[...]) + fb2_ref[...])

    boxes_ref[...] = boxes
    scales_ref[...] = scales
    ctx_ref[...] = ctx
    conf_ref[...] = conf


@jax.jit
def _run(features, W1, b1, ln_g, ln_b, W2, b2,
         lW1, lb1, lW2, lb2, sW1, sb1, sW2, sb2,
         cW1, cb1, cW2, cb2, fW1, fb1, fW2, fb2):
    n, in_dim = features.shape

    wspec = lambda a: pl.BlockSpec(a.shape, lambda i: (0,) * a.ndim)
    row = lambda v: v[None, :]

    weights = (W1, row(b1), row(ln_g), row(ln_b), W2, row(b2),
               lW1, row(lb1), lW2, row(lb2), sW1, row(sb1), sW2, row(sb2),
               cW1, row(cb1), cW2, row(cb2), fW1, row(fb1), fW2, row(fb2))

    out = pl.pallas_call(
        _fused_head_kernel,
        grid=(n // _BLOCK,),
        in_specs=[pl.BlockSpec((_BLOCK, in_dim), lambda i: (i, 0))]
                 + [wspec(w) for w in weights],
        out_specs=[
            pl.BlockSpec((_BLOCK, 4), lambda i: (i, 0)),
            pl.BlockSpec((_BLOCK, 2), lambda i: (i, 0)),
            pl.BlockSpec((_BLOCK, 1), lambda i: (i, 0)),
            pl.BlockSpec((_BLOCK, 1), lambda i: (i, 0)),
        ],
        out_shape=[
            jax.ShapeDtypeStruct((n, 4), jnp.float32),
            jax.ShapeDtypeStruct((n, 2), jnp.float32),
            jax.ShapeDtypeStruct((n, 1), jnp.float32),
            jax.ShapeDtypeStruct((n, 1), jnp.float32),
        ],
        compiler_params=pltpu.CompilerParams(
            dimension_semantics=("parallel",)),
    )(features, *weights)

    return tuple(out)


def kernel(features, W1, b1, ln_g, ln_b, W2, b2, lW1, lb1, lW2, lb2,
           sW1, sb1, sW2, sb2, cW1, cb1, cW2, cb2, fW1, fb1, fW2, fb2):
    return _run(features, W1, b1, ln_g, ln_b, W2, b2,
                lW1, lb1, lW2, lb2, sW1, sb1, sW2, sb2,
                cW1, cb1, cW2, cb2, fW1, fb1, fW2, fb2)


# speedup vs baseline: 1.0612x; 1.0612x over previous
import jax
import jax.numpy as jnp
from jax.experimental import pallas as pl
from jax.experimental.pallas import tpu as pltpu

_BLOCK = 4000


def _fused_head_kernel(feat_ref, w1_ref, b1_ref, gain_ref, beta_ref,
                       w2_ref, b2_ref,
                       lw1_ref, lb1_ref, lw2_ref, lb2_ref,
                       sw1_ref, sb1_ref, sw2_ref, sb2_ref,
                       cw1_ref, cb1_ref, cw2_ref, cb2_ref,
                       fw1_ref, fb1_ref, fw2_ref, fb2_ref,
                       boxes_ref, scales_ref, ctx_ref, conf_ref):
    min_size, max_size = 0.02, 0.1

    def dot(a, b):
        return jnp.dot(a, b, preferred_element_type=jnp.float32)

    x = dot(feat_ref[...], w1_ref[...]) + b1_ref[...]
    mu = jnp.mean(x, axis=-1, keepdims=True)
    var = jnp.mean(jnp.square(x), axis=-1, keepdims=True) - jnp.square(mu)
    x = (x - mu) * jax.lax.rsqrt(var + 1e-5)
    x = jnp.maximum(x * gain_ref[...] + beta_ref[...], 0.0)

    x = dot(x, w2_ref[...]) + b2_ref[...]

    boxes = jax.nn.sigmoid(
        dot(jnp.maximum(dot(x, lw1_ref[...]) + lb1_ref[...], 0.0),
            lw2_ref[...]) + lb2_ref[...])
    scales = jax.nn.sigmoid(
        dot(jnp.maximum(dot(x, sw1_ref[...]) + sb1_ref[...], 0.0),
            sw2_ref[...]) + sb2_ref[...]) * (max_size - min_size) + min_size
    ctx = jax.nn.sigmoid(
        dot(jnp.maximum(dot(x, cw1_ref[...]) + cb1_ref[...], 0.0),
            cw2_ref[...]) + cb2_ref[...])

    combined = jnp.concatenate([boxes, scales, ctx], axis=-1)
    hf = jnp.maximum(
        dot(x, fw1_ref[0:256, :]) + dot(combined, fw1_ref[256:263, :])
        + fb1_ref[...], 0.0)
    conf = jax.nn.sigmoid(dot(hf, fw2_ref[...]) + fb2_ref[...])

    boxes_ref[...] = boxes
    scales_ref[...] = scales
    ctx_ref[...] = ctx
    conf_ref[...] = conf


@jax.jit
def _run(features, W1, b1, ln_g, ln_b, W2, b2,
         lW1, lb1, lW2, lb2, sW1, sb1, sW2, sb2,
         cW1, cb1, cW2, cb2, fW1, fb1, fW2, fb2):
    n, in_dim = features.shape

    wspec = lambda a: pl.BlockSpec(a.shape, lambda i: (0,) * a.ndim)
    row = lambda v: v[None, :]

    weights = (W1, row(b1), row(ln_g), row(ln_b), W2, row(b2),
               lW1, row(lb1), lW2, row(lb2), sW1, row(sb1), sW2, row(sb2),
               cW1, row(cb1), cW2, row(cb2), fW1, row(fb1), fW2, row(fb2))

    out = pl.pallas_call(
        _fused_head_kernel,
        grid=(n // _BLOCK,),
        in_specs=[pl.BlockSpec((_BLOCK, in_dim), lambda i: (i, 0))]
                 + [wspec(w) for w in weights],
        out_specs=[
            pl.BlockSpec((_BLOCK, 4), lambda i: (i, 0)),
            pl.BlockSpec((_BLOCK, 2), lambda i: (i, 0)),
            pl.BlockSpec((_BLOCK, 1), lambda i: (i, 0)),
            pl.BlockSpec((_BLOCK, 1), lambda i: (i, 0)),
        ],
        out_shape=[
            jax.ShapeDtypeStruct((n, 4), jnp.float32),
            jax.ShapeDtypeStruct((n, 2), jnp.float32),
            jax.ShapeDtypeStruct((n, 1), jnp.float32),
            jax.ShapeDtypeStruct((n, 1), jnp.float32),
        ],
        compiler_params=pltpu.CompilerParams(
            dimension_semantics=("parallel",),
            vmem_limit_bytes=100 * 1024 * 1024),
    )(features, *weights)

    return tuple(out)


def kernel(features, W1, b1, ln_g, ln_b, W2, b2, lW1, lb1, lW2, lb2,
           sW1, sb1, sW2, sb2, cW1, cb1, cW2, cb2, fW1, fb1, fW2, fb2):
    return _run(features, W1, b1, ln_g, ln_b, W2, b2,
                lW1, lb1, lW2, lb2, sW1, sb1, sW2, sb2,
                cW1, cb1, cW2, cb2, fW1, fb1, fW2, fb2)
